# baseline (speedup 1.0000x reference)
import functools

import jax
import jax.numpy as jnp
from jax import lax
from jax.experimental import pallas as pl
from jax.experimental.pallas import tpu as pltpu
from jax.experimental.pallas import tpu_sc as plsc

N_NODES = 10000
N_EDGES = 320000
D_FEAT = 128
LANES = 16

NUM_CORES = 2
NUM_SUBCORES = 16
NUM_WORKERS = NUM_CORES * NUM_SUBCORES
E_PER_W = N_EDGES // NUM_WORKERS
CHUNK = 208
E_PAD_W = 10192
NCHUNKS = E_PAD_W // CHUNK
NGROUPS = CHUNK // LANES
OUT_PAD = NUM_WORKERS * E_PAD_W

_mesh = plsc.VectorSubcoreMesh(core_axis_name="c", subcore_axis_name="s")

_GATHER_DNUMS = lax.GatherDimensionNumbers(
    offset_dims=(), collapsed_slice_dims=(0,), start_index_map=(0,))

D_WORDS = D_FEAT // 2
PROWS = 125
NPCH = N_NODES // PROWS
_HI_MASK = -65536


@functools.partial(
    pl.kernel,
    out_type=(jax.ShapeDtypeStruct((N_NODES, D_WORDS), jnp.int32),
              jax.ShapeDtypeStruct((N_NODES, D_WORDS), jnp.int32)),
    mesh=_mesh,
    scratch_types=[
        pltpu.VMEM((2, PROWS, D_FEAT), jnp.float32),
        pltpu.VMEM((2, PROWS, D_WORDS), jnp.int32),
        pltpu.SemaphoreType.DMA,
        pltpu.SemaphoreType.DMA,
    ],
    compiler_params=pltpu.CompilerParams(use_tc_tiling_on_sc=False),
)
def _pack_tables(uf_hbm, if_hbm, upk_hbm, ipk_hbm, in_v, out_v, si0, si1):
    wid = lax.axis_index("s") * NUM_CORES + lax.axis_index("c")
    sem_i = (si0, si1)

    def fire(tin, c, p):
        pltpu.async_copy(tin.at[pl.ds(c * PROWS, PROWS)], in_v.at[p],
                         sem_i[p])

    def pack_chunk(tout, c, p):
        pltpu.make_async_copy(uf_hbm.at[pl.ds(0, PROWS)], in_v.at[p],
                              sem_i[p]).wait()

        def row_body(r, _):
            for b in range(D_FEAT // 32):
                lo = lax.bitcast_convert_type(
                    in_v[p, r, pl.ds(32 * b, LANES)], jnp.int32)
                hi = lax.bitcast_convert_type(
                    in_v[p, r, pl.ds(32 * b + LANES, LANES)], jnp.int32)
                out_v[p, r, pl.ds(LANES * b, LANES)] = (
                    lax.shift_right_logical(lo + 0x8000, 16)
                    | ((hi + 0x8000) & _HI_MASK))
            return 0

        lax.fori_loop(0, PROWS, row_body, 0, unroll=4)
        pltpu.sync_copy(out_v.at[p], tout.at[pl.ds(c * PROWS, PROWS)])

    fire(uf_hbm, wid, 0)
    fire(uf_hbm, wid + 32, 1)
    pack_chunk(upk_hbm, wid, 0)

    @pl.when(wid < 16)
    def _():
        fire(uf_hbm, wid + 64, 0)

    @pl.when(wid >= 16)
    def _():
        fire(if_hbm, wid - 16, 0)

    pack_chunk(upk_hbm, wid + 32, 1)
    fire(if_hbm, wid + 16, 1)

    @pl.when(wid < 16)
    def _():
        pack_chunk(upk_hbm, wid + 64, 0)

    @pl.when(wid >= 16)
    def _():
        pack_chunk(ipk_hbm, wid - 16, 0)

    fire(if_hbm, wid + 48, 0)
    pack_chunk(ipk_hbm, wid + 16, 1)
    pack_chunk(ipk_hbm, wid + 48, 0)


def _lane_take(x, idx):
    return lax.gather(x, idx[:, None], _GATHER_DNUMS, (1,),
                      mode=lax.GatherScatterMode.PROMISE_IN_BOUNDS)


@functools.partial(
    pl.kernel,
    out_type=jax.ShapeDtypeStruct((OUT_PAD,), jnp.float32),
    mesh=_mesh,
    scratch_types=[
        pltpu.VMEM((E_PAD_W,), jnp.int32),
        pltpu.VMEM((E_PAD_W,), jnp.int32),
        pltpu.VMEM((2, CHUNK, D_FEAT // 2), jnp.int32),
        pltpu.VMEM((2, CHUNK, D_FEAT // 2), jnp.int32),
        pltpu.VMEM((CHUNK,), jnp.float32),
        pltpu.VMEM_SHARED((N_NODES, D_WORDS), jnp.int32),
        pltpu.SemaphoreType.DMA,
        pltpu.SemaphoreType.DMA,
        pltpu.SemaphoreType.DMA,
        pltpu.SemaphoreType.DMA,
    ],
    compiler_params=pltpu.CompilerParams(use_tc_tiling_on_sc=False),
)
def _u_dot_v(edge_hbm, ufeat_hbm, ifeat_hbm, out_hbm,
             srcs_v, dsts_v, u_v, v_v, o_v, u_sp, su0, su1, sv0, sv1):
    wid = lax.axis_index("s") * NUM_CORES + lax.axis_index("c")
    sid = lax.axis_index("s")
    base = wid * E_PER_W
    base_out = wid * E_PAD_W
    sem_u = (su0, su1)
    sem_v = (sv0, sv1)

    SLAB = N_NODES // NUM_SUBCORES
    srow = sid * SLAB
    pltpu.sync_copy(ufeat_hbm.at[pl.ds(srow, SLAB)], u_sp.at[pl.ds(srow, SLAB)])
    plsc.subcore_barrier()

    pltpu.sync_copy(edge_hbm.at[0, pl.ds(base, E_PER_W)],
                    srcs_v.at[pl.ds(0, E_PER_W)])
    pltpu.sync_copy(edge_hbm.at[1, pl.ds(base, E_PER_W)],
                    dsts_v.at[pl.ds(0, E_PER_W)])
    zeros16 = jnp.zeros((LANES,), jnp.int32)
    for z in range(E_PER_W, E_PAD_W, LANES):
        srcs_v[pl.ds(z, LANES)] = zeros16
        dsts_v[pl.ds(z, LANES)] = zeros16

    lane_iota = lax.iota(jnp.int32, LANES)
    perms = [(lane_iota ^ sh).astype(jnp.int32) for sh in (8, 4, 2, 1)]

    def fire(c, p):
        off = c * CHUNK
        pltpu.async_copy(u_sp.at[srcs_v.at[pl.ds(off, CHUNK)]],
                         u_v.at[p], sem_u[p])
        pltpu.async_copy(ifeat_hbm.at[dsts_v.at[pl.ds(off, CHUNK)]],
                         v_v.at[p], sem_v[p])

    def wait(p):
        pltpu.make_async_copy(u_sp.at[srcs_v.at[pl.ds(0, CHUNK)]],
                              u_v.at[p], sem_u[p]).wait()
        pltpu.make_async_copy(ifeat_hbm.at[dsts_v.at[pl.ds(0, CHUNK)]],
                              v_v.at[p], sem_v[p]).wait()

    hi_mask = jnp.full((LANES,), -65536, dtype=jnp.int32)

    def bf16_pair(ref, e, b):
        x = ref[e, pl.ds(b * LANES, LANES)]
        lo = lax.bitcast_convert_type(x << 16, jnp.float32)
        hi = lax.bitcast_convert_type(x & hi_mask, jnp.float32)
        return lo, hi

    def edge_dot(p, e):
        acc0 = None
        acc1 = None
        for b in range(D_FEAT // 2 // LANES):
            ua, ub = bf16_pair(u_v.at[p], e, b)
            va, vb = bf16_pair(v_v.at[p], e, b)
            acc0 = ua * va if acc0 is None else acc0 + ua * va
            acc1 = ub * vb if acc1 is None else acc1 + ub * vb
        return acc0 + acc1

    def compute(c, p):
        def group_body(g, _):
            res = jnp.zeros((LANES,), jnp.float32)
            for k in range(LANES):
                acc = edge_dot(p, g * LANES + k)
                for q in perms:
                    acc = acc + _lane_take(acc, q)
                res = jnp.where(lane_iota == k, acc, res)
            o_v[pl.ds(g * LANES, LANES)] = res
            return 0

        lax.fori_loop(0, NGROUPS, group_body, 0)
        pltpu.sync_copy(o_v, out_hbm.at[pl.ds(base_out + c * CHUNK, CHUNK)])

    fire(0, 0)

    def pair_body(t, _):
        c0 = 2 * t
        fire(c0 + 1, 1)
        wait(0)
        compute(c0, 0)
        fire(c0 + 2, 0)
        wait(1)
        compute(c0 + 1, 1)
        return 0

    lax.fori_loop(0, (NCHUNKS - 1) // 2, pair_body, 0)
    wait(0)
    compute(NCHUNKS - 1, 0)


def kernel(ufeat, ifeat, edge_index):
    upk, ipk = _pack_tables(ufeat, ifeat)
    sr_pad = _u_dot_v(edge_index.astype(jnp.int32), upk, ipk)
    sr = sr_pad.reshape(NUM_WORKERS, E_PAD_W)[:, :E_PER_W].reshape(-1)
    return (sr[:, None], ufeat, ifeat)


# speedup vs baseline: 1.8043x; 1.8043x over previous
import functools

import jax
import jax.numpy as jnp
from jax import lax
from jax.experimental import pallas as pl
from jax.experimental.pallas import tpu as pltpu
from jax.experimental.pallas import tpu_sc as plsc

N_NODES = 10000
N_EDGES = 320000
D_FEAT = 128
LANES = 16

NUM_CORES = 2
NUM_SUBCORES = 16
NUM_WORKERS = NUM_CORES * NUM_SUBCORES
E_PER_W = N_EDGES // NUM_WORKERS
CHUNK = 400
NCHUNKS = E_PER_W // CHUNK
NGROUPS = CHUNK // LANES

_mesh = plsc.VectorSubcoreMesh(core_axis_name="c", subcore_axis_name="s")

_GATHER_DNUMS = lax.GatherDimensionNumbers(
    offset_dims=(), collapsed_slice_dims=(0,), start_index_map=(0,))

D_WORDS = D_FEAT // 2
PROWS = 125
NPCH = N_NODES // PROWS
_HI_MASK = -65536


@functools.partial(
    pl.kernel,
    out_type=(jax.ShapeDtypeStruct((N_NODES, D_WORDS), jnp.int32),
              jax.ShapeDtypeStruct((N_NODES, D_WORDS), jnp.int32)),
    mesh=_mesh,
    scratch_types=[
        pltpu.VMEM((2, PROWS, D_FEAT), jnp.float32),
        pltpu.VMEM((2, PROWS, D_WORDS), jnp.int32),
        pltpu.SemaphoreType.DMA,
        pltpu.SemaphoreType.DMA,
    ],
    compiler_params=pltpu.CompilerParams(use_tc_tiling_on_sc=False),
)
def _pack_tables(uf_hbm, if_hbm, upk_hbm, ipk_hbm, in_v, out_v, si0, si1):
    wid = lax.axis_index("s") * NUM_CORES + lax.axis_index("c")
    sem_i = (si0, si1)

    def fire(tin, c, p):
        pltpu.async_copy(tin.at[pl.ds(c * PROWS, PROWS)], in_v.at[p],
                         sem_i[p])

    def pack_chunk(tout, c, p):
        pltpu.make_async_copy(uf_hbm.at[pl.ds(0, PROWS)], in_v.at[p],
                              sem_i[p]).wait()

        def row_body(r, _):
            for b in range(D_FEAT // 32):
                lo = lax.bitcast_convert_type(
                    in_v[p, r, pl.ds(32 * b, LANES)], jnp.int32)
                hi = lax.bitcast_convert_type(
                    in_v[p, r, pl.ds(32 * b + LANES, LANES)], jnp.int32)
                out_v[p, r, pl.ds(LANES * b, LANES)] = (
                    lax.shift_right_logical(lo + 0x8000, 16)
                    | ((hi + 0x8000) & _HI_MASK))
            return 0

        lax.fori_loop(0, PROWS, row_body, 0, unroll=8)
        pltpu.sync_copy(out_v.at[p], tout.at[pl.ds(c * PROWS, PROWS)])

    fire(uf_hbm, wid, 0)
    fire(uf_hbm, wid + 32, 1)
    pack_chunk(upk_hbm, wid, 0)

    @pl.when(wid < 16)
    def _():
        fire(uf_hbm, wid + 64, 0)

    @pl.when(wid >= 16)
    def _():
        fire(if_hbm, wid - 16, 0)

    pack_chunk(upk_hbm, wid + 32, 1)
    fire(if_hbm, wid + 16, 1)

    @pl.when(wid < 16)
    def _():
        pack_chunk(upk_hbm, wid + 64, 0)

    @pl.when(wid >= 16)
    def _():
        pack_chunk(ipk_hbm, wid - 16, 0)

    fire(if_hbm, wid + 48, 0)
    pack_chunk(ipk_hbm, wid + 16, 1)
    pack_chunk(ipk_hbm, wid + 48, 0)


def _lane_take(x, idx):
    return lax.gather(x, idx[:, None], _GATHER_DNUMS, (1,),
                      mode=lax.GatherScatterMode.PROMISE_IN_BOUNDS)


@functools.partial(
    pl.kernel,
    out_type=jax.ShapeDtypeStruct((N_EDGES,), jnp.float32),
    mesh=_mesh,
    scratch_types=[
        pltpu.VMEM((E_PER_W,), jnp.int32),
        pltpu.VMEM((E_PER_W,), jnp.int32),
        pltpu.VMEM((2, CHUNK, D_FEAT // 2), jnp.int32),
        pltpu.VMEM((2, CHUNK, D_FEAT // 2), jnp.int32),
        pltpu.VMEM((CHUNK,), jnp.float32),
        pltpu.SemaphoreType.DMA,
        pltpu.SemaphoreType.DMA,
        pltpu.SemaphoreType.DMA,
        pltpu.SemaphoreType.DMA,
    ],
    compiler_params=pltpu.CompilerParams(use_tc_tiling_on_sc=False),
)
def _u_dot_v(edge_hbm, ufeat_hbm, ifeat_hbm, out_hbm,
             srcs_v, dsts_v, u_v, v_v, o_v, su0, su1, sv0, sv1):
    wid = lax.axis_index("s") * NUM_CORES + lax.axis_index("c")
    base = wid * E_PER_W
    sem_u = (su0, su1)
    sem_v = (sv0, sv1)

    pltpu.sync_copy(edge_hbm.at[0, pl.ds(base, E_PER_W)], srcs_v)
    pltpu.sync_copy(edge_hbm.at[1, pl.ds(base, E_PER_W)], dsts_v)

    lane_iota = lax.iota(jnp.int32, LANES)
    perms = [(lane_iota ^ sh).astype(jnp.int32) for sh in (8, 4, 2, 1)]

    def fire(c, p):
        off = c * CHUNK
        pltpu.async_copy(ufeat_hbm.at[srcs_v.at[pl.ds(off, CHUNK)]],
                         u_v.at[p], sem_u[p])
        pltpu.async_copy(ifeat_hbm.at[dsts_v.at[pl.ds(off, CHUNK)]],
                         v_v.at[p], sem_v[p])

    def wait(p):
        pltpu.make_async_copy(ufeat_hbm.at[srcs_v.at[pl.ds(0, CHUNK)]],
                              u_v.at[p], sem_u[p]).wait()
        pltpu.make_async_copy(ifeat_hbm.at[dsts_v.at[pl.ds(0, CHUNK)]],
                              v_v.at[p], sem_v[p]).wait()

    hi_mask = jnp.full((LANES,), -65536, dtype=jnp.int32)

    def bf16_pair(ref, e, b):
        x = ref[e, pl.ds(b * LANES, LANES)]
        lo = lax.bitcast_convert_type(x << 16, jnp.float32)
        hi = lax.bitcast_convert_type(x & hi_mask, jnp.float32)
        return lo, hi

    def edge_dot(p, e):
        acc0 = None
        acc1 = None
        for b in range(D_FEAT // 2 // LANES):
            ua, ub = bf16_pair(u_v.at[p], e, b)
            va, vb = bf16_pair(v_v.at[p], e, b)
            acc0 = ua * va if acc0 is None else acc0 + ua * va
            acc1 = ub * vb if acc1 is None else acc1 + ub * vb
        return acc0 + acc1

    def compute(c, p):
        def group_body(g, _):
            res = jnp.zeros((LANES,), jnp.float32)
            for k in range(LANES):
                acc = edge_dot(p, g * LANES + k)
                for q in perms:
                    acc = acc + _lane_take(acc, q)
                res = jnp.where(lane_iota == k, acc, res)
            o_v[pl.ds(g * LANES, LANES)] = res
            return 0

        lax.fori_loop(0, NGROUPS, group_body, 0)
        pltpu.sync_copy(o_v, out_hbm.at[pl.ds(base + c * CHUNK, CHUNK)])

    fire(0, 0)

    def pair_body(t, _):
        c0 = 2 * t
        fire(c0 + 1, 1)
        wait(0)
        compute(c0, 0)
        fire(c0 + 2, 0)
        wait(1)
        compute(c0 + 1, 1)
        return 0

    lax.fori_loop(0, (NCHUNKS - 1) // 2, pair_body, 0)
    wait(0)
    compute(NCHUNKS - 1, 0)


def kernel(ufeat, ifeat, edge_index):
    upk, ipk = _pack_tables(ufeat, ifeat)
    sr = _u_dot_v(edge_index.astype(jnp.int32), upk, ipk)
    return (sr[:, None], ufeat, ifeat)
